# baseline (speedup 1.0000x reference)
import functools
import math

import jax
import jax.numpy as jnp
from jax import lax
from jax.experimental import pallas as pl
from jax.experimental.pallas import tpu as pltpu
from jax.experimental.pallas import tpu_sc as plsc

VOCAB = 100000
D = 128
BATCH = 4096
SEQ = 50
NW = 32
K_PARTS = 4
PART_B = BATCH // K_PARTS
PART_ROWS = PART_B * SEQ
B_PER_W = PART_B // NW
ROWS_PER_W = B_PER_W * SEQ
NB = 4
CHUNK = NB * SEQ
GATHER_SPLITS = ((0, 128), (128, 72))
N_CHUNKS = B_PER_W // NB
NBUF = 2
LANES = 16
SCALE = math.sqrt(D)

_mesh = plsc.VectorSubcoreMesh(core_axis_name="c", subcore_axis_name="s")


@functools.partial(
    pl.kernel,
    out_type=jax.ShapeDtypeStruct((PART_ROWS, D), jnp.float32),
    mesh=_mesh,
    scratch_types=[
        pltpu.VMEM((ROWS_PER_W,), jnp.int32),
        pltpu.VMEM((NBUF, CHUNK, D), jnp.float32),
        pltpu.VMEM((NBUF, CHUNK, D), jnp.float32),
    ]
    + [pltpu.SemaphoreType.DMA] * (2 * NBUF),
)
def _gather_part(idx_hbm, table_hbm, out_hbm, idx_v, gbuf, sbuf, *sems):
    gsem = sems[:NBUF]
    ssem = sems[NBUF:]
    wid = lax.axis_index("s") * 2 + lax.axis_index("c")
    base = wid * ROWS_PER_W
    pltpu.sync_copy(idx_hbm.at[pl.ds(base, ROWS_PER_W)], idx_v)

    def gather_descs(c, b):
        return [
            pltpu.make_async_copy(
                table_hbm.at[idx_v.at[pl.ds(c * CHUNK + off, n)]],
                gbuf.at[b, pl.ds(off, n)],
                gsem[b],
            )
            for off, n in GATHER_SPLITS
        ]

    def store_desc(c, b):
        return pltpu.make_async_copy(
            sbuf.at[b], out_hbm.at[pl.ds(base + c * CHUNK, CHUNK)], ssem[b])

    for b in range(NBUF):
        for d in gather_descs(b, b):
            d.start()

    for c in range(N_CHUNKS):
        b = c % NBUF
        for d in gather_descs(c, b):
            d.wait()

        if c >= NBUF:
            store_desc(c - NBUF, b).wait()

        def scale_row(r, _):
            for j in range(D // LANES):
                sl = pl.ds(j * LANES, LANES)
                sbuf[b, r, sl] = gbuf[b, r, sl] * SCALE
            return 0

        lax.fori_loop(0, CHUNK, scale_row, 0)

        if c + NBUF < N_CHUNKS:
            for d in gather_descs(c + NBUF, b):
                d.start()

        store_desc(c, b).start()

    for b in range(NBUF):
        store_desc(N_CHUNKS - NBUF + b, b).wait()


def kernel(input_ids, embed_weight):
    idx = input_ids.reshape(BATCH * SEQ).astype(jnp.int32)
    f = lax.optimization_barrier(jnp.zeros((BATCH, SEQ, D), jnp.float32))
    for k in range(K_PARTS):
        part_idx = lax.slice(
            idx, (k * PART_ROWS,), ((k + 1) * PART_ROWS,))
        raw2d = _gather_part(part_idx, embed_weight)
        part = raw2d.reshape(PART_B, SEQ, D)
        f = lax.dynamic_update_slice(f, part, (k * PART_B, 0, 0))
    return f


# speedup vs baseline: 1.8184x; 1.8184x over previous
import functools
import math

import jax
import jax.numpy as jnp
from jax import lax
from jax.experimental import pallas as pl
from jax.experimental.pallas import tpu as pltpu
from jax.experimental.pallas import tpu_sc as plsc

VOCAB = 100000
D = 128
BATCH = 4096
SEQ = 50
TOTAL = BATCH * SEQ
NW = 32
ROWS_PER_W = TOTAL // NW
B_PER_W = BATCH // NW
NB = 4
CHUNK = NB * SEQ
GATHER_SPLITS = ((0, 128), (128, 72))
N_CHUNKS = B_PER_W // NB
NBUF = 2
N_GROUPS = N_CHUNKS // NBUF
LANES = 16
SCALE = math.sqrt(D)

_mesh = plsc.VectorSubcoreMesh(core_axis_name="c", subcore_axis_name="s")


@functools.partial(
    pl.kernel,
    out_type=jax.ShapeDtypeStruct((BATCH, SEQ, D), jnp.float32),
    mesh=_mesh,
    scratch_types=[
        pltpu.VMEM((ROWS_PER_W,), jnp.int32),
        pltpu.VMEM((NBUF, CHUNK, D), jnp.float32),
        pltpu.VMEM((NBUF, NB, SEQ, D), jnp.float32),
    ]
    + [pltpu.SemaphoreType.DMA] * (2 * NBUF),
)
def _embed_sc(idx_hbm, table_hbm, out_hbm, idx_v, gbuf, sbuf, *sems):
    gsem = sems[:NBUF]
    ssem = sems[NBUF:]
    wid = lax.axis_index("s") * 2 + lax.axis_index("c")
    base = wid * ROWS_PER_W
    b_base = wid * B_PER_W
    pltpu.sync_copy(idx_hbm.at[pl.ds(base, ROWS_PER_W)], idx_v)

    def gather_descs(c, b):
        return [
            pltpu.make_async_copy(
                table_hbm.at[idx_v.at[pl.ds(c * CHUNK + off, n)]],
                gbuf.at[b, pl.ds(off, n)],
                gsem[b],
            )
            for off, n in GATHER_SPLITS
        ]

    def store_desc(c, b):
        return pltpu.make_async_copy(
            sbuf.at[b], out_hbm.at[pl.ds(b_base + c * NB, NB)], ssem[b])

    for b in range(NBUF):
        for d in gather_descs(b, b):
            d.start()

    def group(g, _):
        for b in range(NBUF):
            c = g * NBUF + b
            for d in gather_descs(c, b):
                d.wait()

            @pl.when(g > 0)
            def _():
                store_desc(c - NBUF, b).wait()

            def scale_seq(s, _):
                for bb in range(NB):
                    for j in range(D // LANES):
                        sl = pl.ds(j * LANES, LANES)
                        sbuf[b, bb, s, sl] = gbuf[b, bb * SEQ + s, sl] * SCALE
                return 0

            lax.fori_loop(0, SEQ, scale_seq, 0)

            @pl.when(c + NBUF < N_CHUNKS)
            def _():
                for d in gather_descs(c + NBUF, b):
                    d.start()

            store_desc(c, b).start()
        return 0

    lax.fori_loop(0, N_GROUPS, group, 0)

    for b in range(NBUF):
        store_desc(N_CHUNKS - NBUF + b, b).wait()


def kernel(input_ids, embed_weight):
    idx = input_ids.reshape(TOTAL).astype(jnp.int32)
    return _embed_sc(idx, embed_weight)
